# baseline (speedup 1.0000x reference)
import functools

import jax
import jax.numpy as jnp
from jax import lax
from jax.experimental import pallas as pl
from jax.experimental.pallas import tpu as pltpu
from jax.experimental.pallas import tpu_sc as plsc

_B, _N, _DIM, _K = 4, 4096, 256, 16
_PW = 16
_HID = 64
_RD = 256
_RP = 512
_RM = 128
_RK = _RM * _K
_F32 = jnp.float32
_HI = lax.Precision.HIGHEST

_NSEG = 32
_ROUNDS = 4
_NC, _NS = 2, 16
_NW = _NC * _NS
_CH = 128


def _topk_body(posb_ref, posallT_ref, idx_ref):
    b = pl.program_id(0)
    posb = posb_ref[...]
    posallT = posallT_ref[...]
    xb = jnp.sum(posb * posb, axis=1, keepdims=True)
    xa = jnp.sum(posallT * posallT, axis=0, keepdims=True)
    inner = lax.dot_general(posb.astype(jnp.bfloat16),
                            posallT.astype(jnp.bfloat16),
                            (((1,), (0,)), ((), ())),
                            preferred_element_type=_F32)
    d = xb + xa - 2.0 * inner
    bits = lax.bitcast_convert_type(d, jnp.int32)
    s = jnp.where(bits < 0, bits ^ 0x7FFFFFFF, bits)
    s3 = s.reshape(_RD, _NSEG, 128)
    iota1 = lax.broadcasted_iota(jnp.int32, (_RD, _NSEG, 128), 1)
    p3 = (s3 & ~31) | iota1
    big = jnp.iinfo(jnp.int32).max
    rounds = []
    for r in range(_ROUNDS):
        mr = jnp.min(p3, axis=1, keepdims=True)
        rounds.append(mr)
        if r + 1 < _ROUNDS:
            p3 = jnp.where(iota1 == (mr & 31), big, p3)
    W = jnp.concatenate(rounds, axis=1)
    lane = lax.broadcasted_iota(jnp.int32, (_RD, 128), 1)
    lane3 = lax.broadcasted_iota(jnp.int32, (_RD, _ROUNDS, 128), 2)
    outs = []
    for _ in range(_K):
        m2 = jnp.min(W, axis=1)
        m = jnp.min(m2, axis=1, keepdims=True)
        lstar = jnp.min(jnp.where(m2 == m, lane, _N), axis=1, keepdims=True)
        outs.append((m & 31) * 128 + lstar)
        W = jnp.where((lane3 == lstar[:, :, None]) & (W == m[:, :, None]),
                      big, W)
    idx_ref[...] = jnp.concatenate(outs, axis=1) + b * _N


def _topk(pos_flat, posT):
    grid = (_B, _N // _RD)
    return pl.pallas_call(
        _topk_body,
        grid=grid,
        in_specs=[
            pl.BlockSpec((_RD, _PW), lambda b, r: (b * (_N // _RD) + r, 0)),
            pl.BlockSpec((_PW, _N), lambda b, r: (b, 0)),
        ],
        out_specs=pl.BlockSpec((_RD, _K), lambda b, r: (b * (_N // _RD) + r, 0)),
        out_shape=jax.ShapeDtypeStruct((_B * _N, _K), jnp.int32),
    )(pos_flat, posT)


def _fuse_body(Wq_ref, Wk_ref, Wa1_ref, Wp2_ref, bq_ref, bk_ref, bp2_ref,
               ba1_ref, Wqa_ref, Wka_ref, Wpa_ref, c1_ref):
    Wa1 = Wa1_ref[...]
    mm = functools.partial(jnp.dot, preferred_element_type=_F32, precision=_HI)
    Wqa_ref[...] = mm(Wq_ref[...], Wa1)
    Wka_ref[...] = mm(Wk_ref[...], Wa1)
    Wpa_ref[...] = mm(Wp2_ref[...], Wa1)
    c1_ref[...] = ba1_ref[...] + mm(bq_ref[...] - bk_ref[...] + bp2_ref[...], Wa1)


def _fuse_weights(Wq, Wk, Wa1, W_pos2, bq, bk, b_pos2, ba1):
    return pl.pallas_call(
        _fuse_body,
        out_shape=(
            jax.ShapeDtypeStruct((_DIM, _HID), _F32),
            jax.ShapeDtypeStruct((_DIM, _HID), _F32),
            jax.ShapeDtypeStruct((_DIM, _HID), _F32),
            jax.ShapeDtypeStruct((1, _HID), _F32),
        ),
    )(Wq, Wk, Wa1, W_pos2, bq[None, :], bk[None, :], b_pos2[None, :], ba1[None, :])


_TW = 384


def _dense_body(x_ref, posb_ref, Wv_ref, bv_ref, Wqa_ref, Wka_ref,
                qa_ref, tab_ref):
    xb = x_ref[...]
    mm = functools.partial(jnp.dot, preferred_element_type=_F32)
    qa_ref[...] = mm(xb, Wqa_ref[...])
    v = mm(xb, Wv_ref[...]) + bv_ref[...]
    ka = mm(xb, Wka_ref[...])
    pad = jnp.zeros((_RP, _TW - _DIM - _HID - _PW), _F32)
    tab_ref[...] = jnp.concatenate([v, ka, posb_ref[...], pad], axis=1)


def _dense_tables(xf, pos_flat, Wv, bv, Wqa, Wka):
    grid = ((_B * _N) // _RP,)
    return pl.pallas_call(
        _dense_body,
        grid=grid,
        in_specs=[
            pl.BlockSpec((_RP, _DIM), lambda r: (r, 0)),
            pl.BlockSpec((_RP, _PW), lambda r: (r, 0)),
            pl.BlockSpec((_DIM, _DIM), lambda r: (0, 0)),
            pl.BlockSpec((1, _DIM), lambda r: (0, 0)),
            pl.BlockSpec((_DIM, _HID), lambda r: (0, 0)),
            pl.BlockSpec((_DIM, _HID), lambda r: (0, 0)),
        ],
        out_specs=[
            pl.BlockSpec((_RP, _HID), lambda r: (r, 0)),
            pl.BlockSpec((_RP, _TW), lambda r: (r, 0)),
        ],
        out_shape=[
            jax.ShapeDtypeStruct((_B * _N, _HID), _F32),
            jax.ShapeDtypeStruct((_B * _N, _TW), _F32),
        ],
    )(xf, pos_flat, Wv, bv[None, :], Wqa, Wka)


def _sc_gather(tab, idxf):
    ni = _B * _N * _K
    per_w = ni // _NW
    nch = per_w // _CH
    mesh = plsc.VectorSubcoreMesh(core_axis_name="c", subcore_axis_name="s")

    @functools.partial(
        pl.kernel,
        mesh=mesh,
        out_type=jax.ShapeDtypeStruct((ni, _TW), _F32),
        scratch_types=[
            pltpu.VMEM((_CH,), jnp.int32),
            pltpu.VMEM((_CH, _TW), _F32),
            pltpu.SemaphoreType.DMA,
        ],
    )
    def gk(tab_h, idx_h, out_h, idx_v, buf, sem):
        wid = lax.axis_index("s") * _NC + lax.axis_index("c")

        def body(i, carry):
            base = wid * per_w + i * _CH
            pltpu.sync_copy(idx_h.at[pl.ds(base, _CH)], idx_v)
            pltpu.async_copy(tab_h.at[idx_v], buf, sem).wait()
            pltpu.sync_copy(buf, out_h.at[pl.ds(base, _CH)])
            return carry

        lax.fori_loop(0, nch, body, 0)

    return gk(tab, idxf)


def _main_body(posb_ref, tabg_ref, qa_ref,
               W1p_ref, b1_ref, Wp2_ref, b2_ref, Wpa_ref, c1_ref, Wa2_ref,
               out_ref):
    mm = functools.partial(jnp.dot, preferred_element_type=_F32)
    tabg = tabg_ref[...]
    vg = tabg[:, :_DIM]
    kag = tabg[:, _DIM:_DIM + _HID]
    posg = tabg[:, _DIM + _HID:_DIM + _HID + _PW]
    prel3 = posb_ref[...].reshape(_RM, 1, _PW) - posg.reshape(_RM, _K, _PW)
    prel = prel3.reshape(_RK, _PW)
    h = jnp.maximum(mm(prel, W1p_ref[...]) + b1_ref[...], 0.0)
    pe = mm(h, Wp2_ref[...]) + b2_ref[...]
    qa_rep = jnp.broadcast_to(qa_ref[...].reshape(_RM, 1, _HID),
                              (_RM, _K, _HID)).reshape(_RK, _HID)
    ah = jnp.maximum(qa_rep - kag + mm(h, Wpa_ref[...]) + c1_ref[...],
                     0.0)
    logits = mm(ah, Wa2_ref[...])
    l3 = logits.reshape(_RM, _K, _DIM)
    mx = jnp.max(l3, axis=1, keepdims=True)
    e = jnp.exp(l3 - mx)
    s = jnp.sum(e, axis=1, keepdims=True)
    attn = e / s
    contrib = vg.reshape(_RM, _K, _DIM) + pe.reshape(_RM, _K, _DIM)
    out_ref[...] = jnp.sum(attn * contrib, axis=1)


def _main(pos_flat, tabg, qa, W1p, b_pos1, W_pos2, b_pos2, Wpa, c1, Wa2):
    grid = ((_B * _N) // _RM,)
    return pl.pallas_call(
        _main_body,
        grid=grid,
        in_specs=[
            pl.BlockSpec((_RM, _PW), lambda r: (r, 0)),
            pl.BlockSpec((_RK, _TW), lambda r: (r, 0)),
            pl.BlockSpec((_RM, _HID), lambda r: (r, 0)),
            pl.BlockSpec((_PW, _DIM), lambda r: (0, 0)),
            pl.BlockSpec((1, _DIM), lambda r: (0, 0)),
            pl.BlockSpec((_DIM, _DIM), lambda r: (0, 0)),
            pl.BlockSpec((1, _DIM), lambda r: (0, 0)),
            pl.BlockSpec((_DIM, _HID), lambda r: (0, 0)),
            pl.BlockSpec((1, _HID), lambda r: (0, 0)),
            pl.BlockSpec((_HID, _DIM), lambda r: (0, 0)),
        ],
        out_specs=pl.BlockSpec((_RM, _DIM), lambda r: (r, 0)),
        out_shape=jax.ShapeDtypeStruct((_B * _N, _DIM), _F32),
    )(pos_flat, tabg, qa, W1p, b_pos1[None, :], W_pos2,
      b_pos2[None, :], Wpa, c1, Wa2)


def kernel(x, pos, W_pos1, b_pos1, W_pos2, b_pos2, Wq, bq, Wk, bk, Wv, bv,
           Wa1, ba1, Wa2, ba2):
    xf = x.reshape(_B * _N, _DIM)
    pos_pad = jnp.pad(pos, ((0, 0), (0, 0), (0, _PW - 3)))
    pos_flat = pos_pad.reshape(_B * _N, _PW)
    posT = pos_pad.transpose(0, 2, 1).reshape(_B * _PW, _N)
    W1p = jnp.pad(W_pos1, ((0, _PW - 3), (0, 0)))

    idx = _topk(pos_flat, posT)
    Wqa, Wka, Wpa, c1 = _fuse_weights(Wq, Wk, Wa1, W_pos2, bq, bk, b_pos2, ba1)
    qa, tab = _dense_tables(xf, pos_flat, Wv, bv, Wqa, Wka)
    tabg = _sc_gather(tab, idx.reshape(-1))
    out = _main(pos_flat, tabg, qa,
                W1p, b_pos1, W_pos2, b_pos2, Wpa, c1, Wa2)
    return out.reshape(_B, _N, _DIM)


# speedup vs baseline: 1.6067x; 1.6067x over previous
import functools

import jax
import jax.numpy as jnp
from jax import lax
from jax.experimental import pallas as pl
from jax.experimental.pallas import tpu as pltpu
from jax.experimental.pallas import tpu_sc as plsc

_B, _N, _DIM, _K = 4, 4096, 256, 16
_PW = 16
_HID = 64
_RD = 256
_RP = 512
_RM = 128
_RK = _RM * _K
_F32 = jnp.float32
_HI = lax.Precision.HIGHEST

_NSEG = 32
_ROUNDS = 4
_NC, _NS = 2, 16
_NW = _NC * _NS
_CH = 128


def _topk_body(posb_ref, posallT_ref, idx_ref):
    b = pl.program_id(0)
    posb = posb_ref[...]
    posallT = posallT_ref[...]
    xb = jnp.sum(posb * posb, axis=1, keepdims=True)
    xa = jnp.sum(posallT * posallT, axis=0, keepdims=True)
    inner = lax.dot_general(posb.astype(jnp.bfloat16),
                            posallT.astype(jnp.bfloat16),
                            (((1,), (0,)), ((), ())),
                            preferred_element_type=_F32)
    d = xb + xa - 2.0 * inner
    bits = lax.bitcast_convert_type(d, jnp.int32)
    s = jnp.where(bits < 0, bits ^ 0x7FFFFFFF, bits)
    s3 = s.reshape(_RD, _NSEG, 128)
    iota1 = lax.broadcasted_iota(jnp.int32, (_RD, _NSEG, 128), 1)
    p3 = (s3 & ~31) | iota1
    big = jnp.iinfo(jnp.int32).max
    rounds = []
    for r in range(_ROUNDS):
        mr = jnp.min(p3, axis=1, keepdims=True)
        rounds.append(mr)
        if r + 1 < _ROUNDS:
            p3 = jnp.where(iota1 == (mr & 31), big, p3)
    W = jnp.concatenate([mr.reshape(_RD, 128) for mr in rounds], axis=1)
    wcol = ((W & 31) * 128
            + (lax.broadcasted_iota(jnp.int32, (_RD, _ROUNDS * 128), 1) & 127))
    outs = []
    for _ in range(_K):
        m = jnp.min(W, axis=1, keepdims=True)
        eq = W == m
        outs.append(jnp.min(jnp.where(eq, wcol, 2 * _N), axis=1, keepdims=True))
        W = jnp.where(eq, big, W)
    idx_ref[...] = jnp.concatenate(outs, axis=1) + b * _N


def _topk(pos_flat, posT):
    grid = (_B, _N // _RD)
    return pl.pallas_call(
        _topk_body,
        grid=grid,
        in_specs=[
            pl.BlockSpec((_RD, _PW), lambda b, r: (b * (_N // _RD) + r, 0)),
            pl.BlockSpec((_PW, _N), lambda b, r: (b, 0)),
        ],
        out_specs=pl.BlockSpec((_RD, _K), lambda b, r: (b * (_N // _RD) + r, 0)),
        out_shape=jax.ShapeDtypeStruct((_B * _N, _K), jnp.int32),
    )(pos_flat, posT)


def _fuse_body(Wq_ref, Wk_ref, Wa1_ref, Wp2_ref, bq_ref, bk_ref, bp2_ref,
               ba1_ref, Wqa_ref, Wka_ref, Wpa_ref, c1_ref):
    Wa1 = Wa1_ref[...]
    mm = functools.partial(jnp.dot, preferred_element_type=_F32, precision=_HI)
    Wqa_ref[...] = mm(Wq_ref[...], Wa1)
    Wka_ref[...] = mm(Wk_ref[...], Wa1)
    Wpa_ref[...] = mm(Wp2_ref[...], Wa1)
    c1_ref[...] = ba1_ref[...] + mm(bq_ref[...] - bk_ref[...] + bp2_ref[...], Wa1)


def _fuse_weights(Wq, Wk, Wa1, W_pos2, bq, bk, b_pos2, ba1):
    return pl.pallas_call(
        _fuse_body,
        out_shape=(
            jax.ShapeDtypeStruct((_DIM, _HID), _F32),
            jax.ShapeDtypeStruct((_DIM, _HID), _F32),
            jax.ShapeDtypeStruct((_DIM, _HID), _F32),
            jax.ShapeDtypeStruct((1, _HID), _F32),
        ),
    )(Wq, Wk, Wa1, W_pos2, bq[None, :], bk[None, :], b_pos2[None, :], ba1[None, :])


_TW = 384


def _dense_body(x_ref, posb_ref, Wv_ref, bv_ref, Wqa_ref, Wka_ref,
                qa_ref, tab_ref):
    xb = x_ref[...]
    mm = functools.partial(jnp.dot, preferred_element_type=_F32)
    qa_ref[...] = mm(xb, Wqa_ref[...])
    v = mm(xb, Wv_ref[...]) + bv_ref[...]
    ka = mm(xb, Wka_ref[...])
    pad = jnp.zeros((_RP, _TW - _DIM - _HID - _PW), _F32)
    tab_ref[...] = jnp.concatenate([v, ka, posb_ref[...], pad], axis=1)


def _dense_tables(xf, pos_flat, Wv, bv, Wqa, Wka):
    grid = ((_B * _N) // _RP,)
    return pl.pallas_call(
        _dense_body,
        grid=grid,
        in_specs=[
            pl.BlockSpec((_RP, _DIM), lambda r: (r, 0)),
            pl.BlockSpec((_RP, _PW), lambda r: (r, 0)),
            pl.BlockSpec((_DIM, _DIM), lambda r: (0, 0)),
            pl.BlockSpec((1, _DIM), lambda r: (0, 0)),
            pl.BlockSpec((_DIM, _HID), lambda r: (0, 0)),
            pl.BlockSpec((_DIM, _HID), lambda r: (0, 0)),
        ],
        out_specs=[
            pl.BlockSpec((_RP, _HID), lambda r: (r, 0)),
            pl.BlockSpec((_RP, _TW), lambda r: (r, 0)),
        ],
        out_shape=[
            jax.ShapeDtypeStruct((_B * _N, _HID), _F32),
            jax.ShapeDtypeStruct((_B * _N, _TW), _F32),
        ],
    )(xf, pos_flat, Wv, bv[None, :], Wqa, Wka)


def _sc_gather(tab, idxf):
    ni = _B * _N * _K
    per_w = ni // _NW
    nch = per_w // _CH
    mesh = plsc.VectorSubcoreMesh(core_axis_name="c", subcore_axis_name="s")

    @functools.partial(
        pl.kernel,
        mesh=mesh,
        out_type=jax.ShapeDtypeStruct((ni, _TW), _F32),
        scratch_types=[
            pltpu.VMEM((_CH,), jnp.int32),
            pltpu.VMEM((_CH, _TW), _F32),
            pltpu.SemaphoreType.DMA,
        ],
    )
    def gk(tab_h, idx_h, out_h, idx_v, buf, sem):
        wid = lax.axis_index("s") * _NC + lax.axis_index("c")

        def body(i, carry):
            base = wid * per_w + i * _CH
            pltpu.sync_copy(idx_h.at[pl.ds(base, _CH)], idx_v)
            pltpu.async_copy(tab_h.at[idx_v], buf, sem).wait()
            pltpu.sync_copy(buf, out_h.at[pl.ds(base, _CH)])
            return carry

        lax.fori_loop(0, nch, body, 0)

    return gk(tab, idxf)


def _main_body(posb_ref, tabg_ref, qa_ref,
               W1p_ref, b1_ref, Wp2_ref, b2_ref, Wpa_ref, c1_ref, Wa2_ref,
               out_ref):
    mm = functools.partial(jnp.dot, preferred_element_type=_F32)
    tabg = tabg_ref[...]
    vg = tabg[:, :_DIM]
    kag = tabg[:, _DIM:_DIM + _HID]
    posg = tabg[:, _DIM + _HID:_DIM + _HID + _PW]
    prel3 = posb_ref[...].reshape(_RM, 1, _PW) - posg.reshape(_RM, _K, _PW)
    prel = prel3.reshape(_RK, _PW)
    h = jnp.maximum(mm(prel, W1p_ref[...]) + b1_ref[...], 0.0)
    pe = mm(h, Wp2_ref[...]) + b2_ref[...]
    qa_rep = jnp.broadcast_to(qa_ref[...].reshape(_RM, 1, _HID),
                              (_RM, _K, _HID)).reshape(_RK, _HID)
    ah = jnp.maximum(qa_rep - kag + mm(h, Wpa_ref[...]) + c1_ref[...],
                     0.0)
    logits = mm(ah, Wa2_ref[...])
    l3 = logits.reshape(_RM, _K, _DIM)
    mx = jnp.max(l3, axis=1, keepdims=True)
    e = jnp.exp(l3 - mx)
    s = jnp.sum(e, axis=1, keepdims=True)
    attn = e / s
    contrib = vg.reshape(_RM, _K, _DIM) + pe.reshape(_RM, _K, _DIM)
    out_ref[...] = jnp.sum(attn * contrib, axis=1)


def _main(pos_flat, tabg, qa, W1p, b_pos1, W_pos2, b_pos2, Wpa, c1, Wa2):
    grid = ((_B * _N) // _RM,)
    return pl.pallas_call(
        _main_body,
        grid=grid,
        in_specs=[
            pl.BlockSpec((_RM, _PW), lambda r: (r, 0)),
            pl.BlockSpec((_RK, _TW), lambda r: (r, 0)),
            pl.BlockSpec((_RM, _HID), lambda r: (r, 0)),
            pl.BlockSpec((_PW, _DIM), lambda r: (0, 0)),
            pl.BlockSpec((1, _DIM), lambda r: (0, 0)),
            pl.BlockSpec((_DIM, _DIM), lambda r: (0, 0)),
            pl.BlockSpec((1, _DIM), lambda r: (0, 0)),
            pl.BlockSpec((_DIM, _HID), lambda r: (0, 0)),
            pl.BlockSpec((1, _HID), lambda r: (0, 0)),
            pl.BlockSpec((_HID, _DIM), lambda r: (0, 0)),
        ],
        out_specs=pl.BlockSpec((_RM, _DIM), lambda r: (r, 0)),
        out_shape=jax.ShapeDtypeStruct((_B * _N, _DIM), _F32),
    )(pos_flat, tabg, qa, W1p, b_pos1[None, :], W_pos2,
      b_pos2[None, :], Wpa, c1, Wa2)


def kernel(x, pos, W_pos1, b_pos1, W_pos2, b_pos2, Wq, bq, Wk, bk, Wv, bv,
           Wa1, ba1, Wa2, ba2):
    xf = x.reshape(_B * _N, _DIM)
    pos_pad = jnp.pad(pos, ((0, 0), (0, 0), (0, _PW - 3)))
    pos_flat = pos_pad.reshape(_B * _N, _PW)
    posT = pos_pad.transpose(0, 2, 1).reshape(_B * _PW, _N)
    W1p = jnp.pad(W_pos1, ((0, _PW - 3), (0, 0)))

    idx = _topk(pos_flat, posT)
    Wqa, Wka, Wpa, c1 = _fuse_weights(Wq, Wk, Wa1, W_pos2, bq, bk, b_pos2, ba1)
    qa, tab = _dense_tables(xf, pos_flat, Wv, bv, Wqa, Wka)
    tabg = _sc_gather(tab, idx.reshape(-1))
    out = _main(pos_flat, tabg, qa,
                W1p, b_pos1, W_pos2, b_pos2, Wpa, c1, Wa2)
    return out.reshape(_B, _N, _DIM)


# speedup vs baseline: 1.6584x; 1.0322x over previous
import functools

import jax
import jax.numpy as jnp
from jax import lax
from jax.experimental import pallas as pl
from jax.experimental.pallas import tpu as pltpu
from jax.experimental.pallas import tpu_sc as plsc

_B, _N, _DIM, _K = 4, 4096, 256, 16
_PW = 16
_HID = 64
_RD = 512
_RP = 512
_RM = 256
_RK = _RM * _K
_F32 = jnp.float32
_HI = lax.Precision.HIGHEST

_NSEG = 32
_ROUNDS = 4
_NC, _NS = 2, 16
_NW = _NC * _NS
_CH = 128


def _topk_body(posb_ref, posallT_ref, idx_ref):
    b = pl.program_id(0)
    posb = posb_ref[...]
    posallT = posallT_ref[...]
    xb = jnp.sum(posb * posb, axis=1, keepdims=True)
    xa = jnp.sum(posallT * posallT, axis=0, keepdims=True)
    inner = lax.dot_general(posb.astype(jnp.bfloat16),
                            posallT.astype(jnp.bfloat16),
                            (((1,), (0,)), ((), ())),
                            preferred_element_type=_F32)
    d = xb + xa - 2.0 * inner
    bits = lax.bitcast_convert_type(d, jnp.int32)
    s = jnp.where(bits < 0, bits ^ 0x7FFFFFFF, bits)
    s3 = s.reshape(_RD, _NSEG, 128)
    iota1 = lax.broadcasted_iota(jnp.int32, (_RD, _NSEG, 128), 1)
    p3 = (s3 & ~31) | iota1
    big = jnp.iinfo(jnp.int32).max
    rounds = []
    for r in range(_ROUNDS):
        mr = jnp.min(p3, axis=1, keepdims=True)
        rounds.append(mr)
        if r + 1 < _ROUNDS:
            p3 = jnp.where(iota1 == (mr & 31), big, p3)
    W = jnp.concatenate([mr.reshape(_RD, 128) for mr in rounds], axis=1)
    wcol = ((W & 31) * 128
            + (lax.broadcasted_iota(jnp.int32, (_RD, _ROUNDS * 128), 1) & 127))
    outs = []
    for _ in range(_K):
        m = jnp.min(W, axis=1, keepdims=True)
        eq = W == m
        outs.append(jnp.min(jnp.where(eq, wcol, 2 * _N), axis=1, keepdims=True))
        W = jnp.where(eq, big, W)
    idx_ref[...] = jnp.concatenate(outs, axis=1) + b * _N


def _topk(pos_flat, posT):
    grid = (_B, _N // _RD)
    return pl.pallas_call(
        _topk_body,
        grid=grid,
        in_specs=[
            pl.BlockSpec((_RD, _PW), lambda b, r: (b * (_N // _RD) + r, 0)),
            pl.BlockSpec((_PW, _N), lambda b, r: (b, 0)),
        ],
        out_specs=pl.BlockSpec((_RD, _K), lambda b, r: (b * (_N // _RD) + r, 0)),
        out_shape=jax.ShapeDtypeStruct((_B * _N, _K), jnp.int32),
    )(pos_flat, posT)


def _fuse_body(Wq_ref, Wk_ref, Wa1_ref, Wp2_ref, bq_ref, bk_ref, bp2_ref,
               ba1_ref, Wqa_ref, Wka_ref, Wpa_ref, c1_ref):
    Wa1 = Wa1_ref[...]
    mm = functools.partial(jnp.dot, preferred_element_type=_F32, precision=_HI)
    Wqa_ref[...] = mm(Wq_ref[...], Wa1)
    Wka_ref[...] = mm(Wk_ref[...], Wa1)
    Wpa_ref[...] = mm(Wp2_ref[...], Wa1)
    c1_ref[...] = ba1_ref[...] + mm(bq_ref[...] - bk_ref[...] + bp2_ref[...], Wa1)


def _fuse_weights(Wq, Wk, Wa1, W_pos2, bq, bk, b_pos2, ba1):
    return pl.pallas_call(
        _fuse_body,
        out_shape=(
            jax.ShapeDtypeStruct((_DIM, _HID), _F32),
            jax.ShapeDtypeStruct((_DIM, _HID), _F32),
            jax.ShapeDtypeStruct((_DIM, _HID), _F32),
            jax.ShapeDtypeStruct((1, _HID), _F32),
        ),
    )(Wq, Wk, Wa1, W_pos2, bq[None, :], bk[None, :], b_pos2[None, :], ba1[None, :])


_TW = 384


def _dense_body(x_ref, posb_ref, Wv_ref, bv_ref, Wqa_ref, Wka_ref,
                qa_ref, tab_ref):
    xb = x_ref[...]
    mm = functools.partial(jnp.dot, preferred_element_type=_F32)
    qa_ref[...] = mm(xb, Wqa_ref[...])
    v = mm(xb, Wv_ref[...]) + bv_ref[...]
    ka = mm(xb, Wka_ref[...])
    pad = jnp.zeros((_RP, _TW - _DIM - _HID - _PW), _F32)
    tab_ref[...] = jnp.concatenate([v, ka, posb_ref[...], pad], axis=1)


def _dense_tables(xf, pos_flat, Wv, bv, Wqa, Wka):
    grid = ((_B * _N) // _RP,)
    return pl.pallas_call(
        _dense_body,
        grid=grid,
        in_specs=[
            pl.BlockSpec((_RP, _DIM), lambda r: (r, 0)),
            pl.BlockSpec((_RP, _PW), lambda r: (r, 0)),
            pl.BlockSpec((_DIM, _DIM), lambda r: (0, 0)),
            pl.BlockSpec((1, _DIM), lambda r: (0, 0)),
            pl.BlockSpec((_DIM, _HID), lambda r: (0, 0)),
            pl.BlockSpec((_DIM, _HID), lambda r: (0, 0)),
        ],
        out_specs=[
            pl.BlockSpec((_RP, _HID), lambda r: (r, 0)),
            pl.BlockSpec((_RP, _TW), lambda r: (r, 0)),
        ],
        out_shape=[
            jax.ShapeDtypeStruct((_B * _N, _HID), _F32),
            jax.ShapeDtypeStruct((_B * _N, _TW), _F32),
        ],
    )(xf, pos_flat, Wv, bv[None, :], Wqa, Wka)


def _sc_gather(tab, idxf):
    ni = _B * _N * _K
    per_w = ni // _NW
    nch = per_w // _CH
    mesh = plsc.VectorSubcoreMesh(core_axis_name="c", subcore_axis_name="s")

    @functools.partial(
        pl.kernel,
        mesh=mesh,
        out_type=jax.ShapeDtypeStruct((ni, _TW), _F32),
        scratch_types=[
            pltpu.VMEM((_CH,), jnp.int32),
            pltpu.VMEM((_CH, _TW), _F32),
            pltpu.SemaphoreType.DMA,
        ],
    )
    def gk(tab_h, idx_h, out_h, idx_v, buf, sem):
        wid = lax.axis_index("s") * _NC + lax.axis_index("c")

        def body(i, carry):
            base = wid * per_w + i * _CH
            pltpu.sync_copy(idx_h.at[pl.ds(base, _CH)], idx_v)
            pltpu.async_copy(tab_h.at[idx_v], buf, sem).wait()
            pltpu.sync_copy(buf, out_h.at[pl.ds(base, _CH)])
            return carry

        lax.fori_loop(0, nch, body, 0)

    return gk(tab, idxf)


def _main_body(posb_ref, tabg_ref, qa_ref,
               W1p_ref, b1_ref, Wp2_ref, b2_ref, Wpa_ref, c1_ref, Wa2_ref,
               out_ref):
    mm = functools.partial(jnp.dot, preferred_element_type=_F32)
    tabg = tabg_ref[...]
    vg = tabg[:, :_DIM]
    kag = tabg[:, _DIM:_DIM + _HID]
    posg = tabg[:, _DIM + _HID:_DIM + _HID + _PW]
    prel3 = posb_ref[...].reshape(_RM, 1, _PW) - posg.reshape(_RM, _K, _PW)
    prel = prel3.reshape(_RK, _PW)
    h = jnp.maximum(mm(prel, W1p_ref[...]) + b1_ref[...], 0.0)
    pe = mm(h, Wp2_ref[...]) + b2_ref[...]
    qa_rep = jnp.broadcast_to(qa_ref[...].reshape(_RM, 1, _HID),
                              (_RM, _K, _HID)).reshape(_RK, _HID)
    ah = jnp.maximum(qa_rep - kag + mm(h, Wpa_ref[...]) + c1_ref[...],
                     0.0)
    logits = mm(ah, Wa2_ref[...])
    l3 = logits.reshape(_RM, _K, _DIM)
    mx = jnp.max(l3, axis=1, keepdims=True)
    e = jnp.exp(l3 - mx)
    s = jnp.sum(e, axis=1, keepdims=True)
    attn = e / s
    contrib = vg.reshape(_RM, _K, _DIM) + pe.reshape(_RM, _K, _DIM)
    out_ref[...] = jnp.sum(attn * contrib, axis=1)


def _main(pos_flat, tabg, qa, W1p, b_pos1, W_pos2, b_pos2, Wpa, c1, Wa2):
    grid = ((_B * _N) // _RM,)
    return pl.pallas_call(
        _main_body,
        grid=grid,
        in_specs=[
            pl.BlockSpec((_RM, _PW), lambda r: (r, 0)),
            pl.BlockSpec((_RK, _TW), lambda r: (r, 0)),
            pl.BlockSpec((_RM, _HID), lambda r: (r, 0)),
            pl.BlockSpec((_PW, _DIM), lambda r: (0, 0)),
            pl.BlockSpec((1, _DIM), lambda r: (0, 0)),
            pl.BlockSpec((_DIM, _DIM), lambda r: (0, 0)),
            pl.BlockSpec((1, _DIM), lambda r: (0, 0)),
            pl.BlockSpec((_DIM, _HID), lambda r: (0, 0)),
            pl.BlockSpec((1, _HID), lambda r: (0, 0)),
            pl.BlockSpec((_HID, _DIM), lambda r: (0, 0)),
        ],
        out_specs=pl.BlockSpec((_RM, _DIM), lambda r: (r, 0)),
        out_shape=jax.ShapeDtypeStruct((_B * _N, _DIM), _F32),
    )(pos_flat, tabg, qa, W1p, b_pos1[None, :], W_pos2,
      b_pos2[None, :], Wpa, c1, Wa2)


def kernel(x, pos, W_pos1, b_pos1, W_pos2, b_pos2, Wq, bq, Wk, bk, Wv, bv,
           Wa1, ba1, Wa2, ba2):
    xf = x.reshape(_B * _N, _DIM)
    pos_pad = jnp.pad(pos, ((0, 0), (0, 0), (0, _PW - 3)))
    pos_flat = pos_pad.reshape(_B * _N, _PW)
    posT = pos_pad.transpose(0, 2, 1).reshape(_B * _PW, _N)
    W1p = jnp.pad(W_pos1, ((0, _PW - 3), (0, 0)))

    idx = _topk(pos_flat, posT)
    Wqa, Wka, Wpa, c1 = _fuse_weights(Wq, Wk, Wa1, W_pos2, bq, bk, b_pos2, ba1)
    qa, tab = _dense_tables(xf, pos_flat, Wv, bv, Wqa, Wka)
    tabg = _sc_gather(tab, idx.reshape(-1))
    out = _main(pos_flat, tabg, qa,
                W1p, b_pos1, W_pos2, b_pos2, Wpa, c1, Wa2)
    return out.reshape(_B, _N, _DIM)
